# Initial kernel scaffold; baseline (speedup 1.0000x reference)
import jax
import jax.numpy as jnp
from jax.experimental import pallas as pl


def kernel(obj_vecs, rel_vecs, edge_index, W_obj, b_obj, W_rel, b_rel):
    raise NotImplementedError("write your pallas kernel here")


# speedup vs baseline: 4.0068x; 4.0068x over previous
import functools

import jax
import jax.numpy as jnp
from jax.experimental import pallas as pl
from jax.experimental.pallas import tpu as pltpu
from jax.experimental.pallas import tpu_sc as plsc

_NC = 2
_NS = 16
_NW = _NC * _NS


def _linear_body(x_ref, w_ref, b_ref, o_ref):
    o_ref[...] = jax.lax.dot_general(
        x_ref[...], w_ref[...],
        dimension_numbers=(((1,), (1,)), ((), ())),
        preferred_element_type=jnp.float32,
    ) + b_ref[...]


def _linear(x, w, b2d, blk):
    m, k = x.shape
    dout = w.shape[0]
    return pl.pallas_call(
        _linear_body,
        grid=(m // blk,),
        in_specs=[
            pl.BlockSpec((blk, k), lambda i: (i, 0)),
            pl.BlockSpec((dout, k), lambda i: (0, 0)),
            pl.BlockSpec((1, dout), lambda i: (0, 0)),
        ],
        out_specs=pl.BlockSpec((blk, dout), lambda i: (i, 0)),
        out_shape=jax.ShapeDtypeStruct((m, dout), jnp.float32),
    )(x, w, b2d)


def _combine_body(a_ref, b_ref, o_ref):
    o_ref[...] = jnp.maximum(a_ref[...] + b_ref[...], 0.0)


def _make_edge_kernel(n_nodes, n_edges, d):
    e_per_tile = n_edges // _NW
    eb = 80
    pub_tiles = 10
    rows_per_pub = n_nodes // pub_tiles
    zrows = 40

    mesh = plsc.VectorSubcoreMesh(core_axis_name="c", subcore_axis_name="s")

    @functools.partial(
        pl.kernel,
        out_type=jax.ShapeDtypeStruct((_NC * n_nodes, d), jnp.float32),
        mesh=mesh,
        scratch_types=[
            pltpu.VMEM((eb,), jnp.int32),
            pltpu.VMEM((eb,), jnp.int32),
            pltpu.VMEM((eb, d), jnp.float32),
            pltpu.VMEM((eb, d), jnp.float32),
            pltpu.VMEM((eb, d), jnp.float32),
            pltpu.VMEM((zrows, d), jnp.float32),
            pltpu.VMEM_SHARED((n_nodes, d), jnp.float32),
            pltpu.SemaphoreType.DMA,
            pltpu.SemaphoreType.DMA,
            pltpu.SemaphoreType.DMA,
        ],
    )
    def edge_kernel(p_hbm, r_hbm, src_hbm, dst_hbm, out_hbm,
                    srcv, dstv, rv, psv, pdv, zv, acc,
                    sem1, sem2, sem3):
        c = jax.lax.axis_index("c")
        s = jax.lax.axis_index("s")

        @pl.when(s < pub_tiles)
        def _():
            @pl.loop(0, zrows)
            def _(i):
                for j in range(0, d, 16):
                    zv[i, pl.ds(j, 16)] = jnp.zeros((16,), jnp.float32)

            @pl.loop(0, rows_per_pub, step=zrows)
            def _(k):
                pltpu.sync_copy(zv, acc.at[pl.ds(s * rows_per_pub + k, zrows)])

        plsc.subcore_barrier()

        base = (c * _NS + s) * e_per_tile

        @pl.loop(0, e_per_tile, step=eb)
        def _(e0):
            g0 = base + e0
            pltpu.sync_copy(src_hbm.at[pl.ds(g0, eb)], srcv)
            pltpu.sync_copy(dst_hbm.at[pl.ds(g0, eb)], dstv)
            cp1 = pltpu.async_copy(p_hbm.at[srcv], psv, sem1)
            cp2 = pltpu.async_copy(p_hbm.at[dstv], pdv, sem2)
            cp3 = pltpu.async_copy(r_hbm.at[pl.ds(g0, eb)], rv, sem3)
            cp1.wait()
            cp2.wait()
            cp3.wait()

            @pl.loop(0, eb)
            def _(e):
                for j in range(0, d, 16):
                    sl = pl.ds(j, 16)
                    rv[e, sl] = jnp.maximum(
                        rv[e, sl] + psv[e, sl] + pdv[e, sl], 0.0)

            pltpu.sync_copy(rv, acc.at[dstv], add=True)

        plsc.subcore_barrier()

        @pl.when(s < pub_tiles)
        def _():
            pltpu.sync_copy(
                acc.at[pl.ds(s * rows_per_pub, rows_per_pub)],
                out_hbm.at[pl.ds(c * n_nodes + s * rows_per_pub,
                                 rows_per_pub)],
            )

    return edge_kernel


def kernel(obj_vecs, rel_vecs, edge_index, W_obj, b_obj, W_rel, b_rel):
    n_nodes, d = obj_vecs.shape
    n_edges = rel_vecs.shape[0]

    src = edge_index[:, 0].astype(jnp.int32)
    dst = edge_index[:, 1].astype(jnp.int32)

    p = _linear(obj_vecs, W_obj, b_obj.reshape(1, -1), blk=2000)
    r = _linear(rel_vecs, W_rel, b_rel.reshape(1, -1), blk=2560)

    partials = _make_edge_kernel(n_nodes, n_edges, d)(p, r, src, dst)

    blk = 2000
    out = pl.pallas_call(
        _combine_body,
        grid=(n_nodes // blk,),
        in_specs=[
            pl.BlockSpec((blk, d), lambda i: (i, 0)),
            pl.BlockSpec((blk, d), lambda i: (i + n_nodes // blk, 0)),
        ],
        out_specs=pl.BlockSpec((blk, d), lambda i: (i, 0)),
        out_shape=jax.ShapeDtypeStruct((n_nodes, d), jnp.float32),
    )(partials, partials)
    return out


# speedup vs baseline: 6.2423x; 1.5580x over previous
import functools

import jax
import jax.numpy as jnp
from jax.experimental import pallas as pl
from jax.experimental.pallas import tpu as pltpu
from jax.experimental.pallas import tpu_sc as plsc

_NC = 2
_NS = 16
_NW = _NC * _NS


def _linear_body(x_ref, w_ref, b_ref, o_ref):
    o_ref[...] = jax.lax.dot_general(
        x_ref[...], w_ref[...],
        dimension_numbers=(((1,), (1,)), ((), ())),
        preferred_element_type=jnp.float32,
    ) + b_ref[...]


def _linear(x, w, b2d, blk):
    m, k = x.shape
    dout = w.shape[0]
    return pl.pallas_call(
        _linear_body,
        grid=(m // blk,),
        in_specs=[
            pl.BlockSpec((blk, k), lambda i: (i, 0)),
            pl.BlockSpec((dout, k), lambda i: (0, 0)),
            pl.BlockSpec((1, dout), lambda i: (0, 0)),
        ],
        out_specs=pl.BlockSpec((blk, dout), lambda i: (i, 0)),
        out_shape=jax.ShapeDtypeStruct((m, dout), jnp.float32),
    )(x, w, b2d)


def _combine_body(a_ref, b_ref, o_ref):
    o_ref[...] = jnp.maximum(a_ref[...] + b_ref[...], 0.0)


def _make_edge_kernel(n_nodes, n_edges, d):
    e_per_tile = n_edges // _NW
    eb = 40
    nblk = e_per_tile // eb
    pub_tiles = 10
    rows_per_pub = n_nodes // pub_tiles

    mesh = plsc.VectorSubcoreMesh(core_axis_name="c", subcore_axis_name="s")

    @functools.partial(
        pl.kernel,
        out_type=jax.ShapeDtypeStruct((_NC * n_nodes, d), jnp.float32),
        mesh=mesh,
        scratch_types=[
            pltpu.VMEM((e_per_tile,), jnp.int32),
            pltpu.VMEM((e_per_tile,), jnp.int32),
            pltpu.VMEM((eb, d), jnp.float32),
            pltpu.VMEM((eb, d), jnp.float32),
            pltpu.VMEM((eb, d), jnp.float32),
            pltpu.VMEM((eb, d), jnp.float32),
            pltpu.VMEM((eb, d), jnp.float32),
            pltpu.VMEM((eb, d), jnp.float32),
            pltpu.VMEM_SHARED((n_nodes, d), jnp.float32),
            pltpu.SemaphoreType.DMA,
            pltpu.SemaphoreType.DMA,
            pltpu.SemaphoreType.DMA,
            pltpu.SemaphoreType.DMA,
            pltpu.SemaphoreType.DMA,
            pltpu.SemaphoreType.DMA,
            pltpu.SemaphoreType.DMA,
            pltpu.SemaphoreType.DMA,
        ],
    )
    def edge_kernel(p_hbm, r_hbm, src_hbm, dst_hbm, out_hbm,
                    srcv, dstv, rv0, rv1, ps0, ps1, pd0, pd1, acc,
                    psem0, psem1, dsem0, dsem1, rsem0, rsem1,
                    ssem0, ssem1):
        c = jax.lax.axis_index("c")
        s = jax.lax.axis_index("s")
        wid = c * _NS + s
        base = wid * e_per_tile

        bufs = ((rv0, ps0, pd0, psem0, dsem0, rsem0, ssem0),
                (rv1, ps1, pd1, psem1, dsem1, rsem1, ssem1))

        @pl.when(s < pub_tiles)
        def _():
            @pl.loop(0, eb)
            def _(i):
                for j in range(0, d, 16):
                    ps0[i, pl.ds(j, 16)] = jnp.zeros((16,), jnp.float32)

            @pl.loop(0, rows_per_pub, step=eb)
            def _(k):
                pltpu.sync_copy(ps0, acc.at[pl.ds(s * rows_per_pub + k, eb)])

        plsc.subcore_barrier()

        pltpu.sync_copy(src_hbm.at[pl.ds(base, e_per_tile)], srcv)
        pltpu.sync_copy(dst_hbm.at[pl.ds(base, e_per_tile)], dstv)

        def fetch(g, par):
            rv, ps, pd, psem, dsem, rsem, _ = bufs[par]
            pltpu.async_copy(
                p_hbm.at[srcv.at[pl.ds(g * eb, eb)]], ps, psem)
            pltpu.async_copy(
                p_hbm.at[dstv.at[pl.ds(g * eb, eb)]], pd, dsem)
            pltpu.async_copy(r_hbm.at[pl.ds(base + g * eb, eb)], rv, rsem)

        def wait_fetch(g, par):
            rv, ps, pd, psem, dsem, rsem, _ = bufs[par]
            pltpu.make_async_copy(
                p_hbm.at[srcv.at[pl.ds(g * eb, eb)]], ps, psem).wait()
            pltpu.make_async_copy(
                p_hbm.at[dstv.at[pl.ds(g * eb, eb)]], pd, dsem).wait()
            pltpu.make_async_copy(
                r_hbm.at[pl.ds(base + g * eb, eb)], rv, rsem).wait()

        fetch(0, 0)

        @pl.loop(0, nblk, step=2)
        def _(g0):
            for par in (0, 1):
                g = g0 + par
                rv, ps, pd, _, _, _, ssem = bufs[par]
                orv, _, _, _, _, _, ossem = bufs[1 - par]

                @pl.when(g >= 1)
                def _():
                    pltpu.make_async_copy(
                        orv, acc.at[dstv.at[pl.ds((g - 1) * eb, eb)]],
                        ossem).wait()

                @pl.when(g + 1 < nblk)
                def _():
                    fetch(g + 1, 1 - par)

                wait_fetch(g, par)

                @pl.loop(0, eb)
                def _(e):
                    for j in range(0, d, 16):
                        sl = pl.ds(j, 16)
                        rv[e, sl] = jnp.maximum(
                            rv[e, sl] + ps[e, sl] + pd[e, sl], 0.0)

                pltpu.async_copy(
                    rv, acc.at[dstv.at[pl.ds(g * eb, eb)]], ssem, add=True)

        pltpu.make_async_copy(
            rv1, acc.at[dstv.at[pl.ds((nblk - 1) * eb, eb)]], ssem1).wait()

        plsc.subcore_barrier()

        @pl.when(s < pub_tiles)
        def _():
            pltpu.sync_copy(
                acc.at[pl.ds(s * rows_per_pub, rows_per_pub)],
                out_hbm.at[pl.ds(c * n_nodes + s * rows_per_pub,
                                 rows_per_pub)],
            )

    return edge_kernel


def kernel(obj_vecs, rel_vecs, edge_index, W_obj, b_obj, W_rel, b_rel):
    n_nodes, d = obj_vecs.shape
    n_edges = rel_vecs.shape[0]

    src = edge_index[:, 0].astype(jnp.int32)
    dst = edge_index[:, 1].astype(jnp.int32)

    p = _linear(obj_vecs, W_obj, b_obj.reshape(1, -1), blk=2000)
    r = _linear(rel_vecs, W_rel, b_rel.reshape(1, -1), blk=2560)

    partials = _make_edge_kernel(n_nodes, n_edges, d)(p, r, src, dst)

    blk = 2000
    out = pl.pallas_call(
        _combine_body,
        grid=(n_nodes // blk,),
        in_specs=[
            pl.BlockSpec((blk, d), lambda i: (i, 0)),
            pl.BlockSpec((blk, d), lambda i: (i + n_nodes // blk, 0)),
        ],
        out_specs=pl.BlockSpec((blk, d), lambda i: (i, 0)),
        out_shape=jax.ShapeDtypeStruct((n_nodes, d), jnp.float32),
    )(partials, partials)
    return out
